# Initial kernel scaffold; baseline (speedup 1.0000x reference)
import functools

import jax
import jax.numpy as jnp
from jax import lax
from jax.experimental import pallas as pl
from jax.experimental.pallas import tpu as pltpu
from jax.experimental.pallas import tpu_sc as plsc

NE = 320000
NN = 10000
D = 128

NC = 2
NS = 16
NW = NC * NS
EPW = NE // NW
G = 80
CH = 400
NCHUNK = EPW // CH
IROWS = EPW // G
RPS = NN // NS

_mesh = plsc.VectorSubcoreMesh(core_axis_name="c", subcore_axis_name="s")


def _scatter_body(eh_hbm, dst_hbm, zero_hbm, parts_hbm, idx_v, ebuf, mail_sh):
    cid = lax.axis_index("c")
    sid = lax.axis_index("s")
    wid = cid * NS + sid

    pltpu.sync_copy(zero_hbm.at[pl.ds(sid * RPS, RPS)],
                    mail_sh.at[pl.ds(sid * RPS, RPS)])
    plsc.subcore_barrier()

    pltpu.sync_copy(dst_hbm.at[pl.ds(wid * IROWS, IROWS)], idx_v)

    ebase = wid * EPW

    def chunk(i, carry):
        pltpu.sync_copy(eh_hbm.at[pl.ds(ebase + i * CH, CH)], ebuf)
        for b in range(CH // G):
            pltpu.sync_copy(ebuf.at[pl.ds(b * G, G)],
                            mail_sh.at[idx_v.at[i * (CH // G) + b]],
                            add=True)
        return carry

    lax.fori_loop(0, NCHUNK, chunk, 0)
    plsc.subcore_barrier()

    pltpu.sync_copy(mail_sh.at[pl.ds(sid * RPS, RPS)],
                    parts_hbm.at[cid, pl.ds(sid * RPS, RPS)])


_scatter = pl.kernel(
    _scatter_body,
    out_type=jax.ShapeDtypeStruct((NC, NN, D), jnp.float32),
    mesh=_mesh,
    scratch_types=[
        pltpu.VMEM((IROWS, G), jnp.int32),
        pltpu.VMEM((CH, D), jnp.float32),
        pltpu.VMEM_SHARED((NN, D), jnp.float32),
    ],
)


def _gather_body(mw_hbm, src_hbm, out_hbm, idx_v, gbuf, sem):
    cid = lax.axis_index("c")
    sid = lax.axis_index("s")
    wid = cid * NS + sid

    pltpu.sync_copy(src_hbm.at[pl.ds(wid * IROWS, IROWS)], idx_v)
    ebase = wid * EPW

    def chunk(i, carry):
        descs = [
            pltpu.async_copy(mw_hbm.at[idx_v.at[i * (CH // G) + b]],
                             gbuf.at[pl.ds(b * G, G)], sem)
            for b in range(CH // G)
        ]
        for d in descs:
            d.wait()
        pltpu.sync_copy(gbuf, out_hbm.at[pl.ds(ebase + i * CH, CH)])
        return carry

    lax.fori_loop(0, NCHUNK, chunk, 0)


_gather = pl.kernel(
    _gather_body,
    out_type=jax.ShapeDtypeStruct((NE, D), jnp.float32),
    mesh=_mesh,
    scratch_types=[
        pltpu.VMEM((IROWS, G), jnp.int32),
        pltpu.VMEM((CH, D), jnp.float32),
        pltpu.SemaphoreType.DMA,
    ],
)


def _mailw_body(parts_ref, w_ref, o_ref):
    p = parts_ref[0] + parts_ref[1]
    o_ref[...] = jnp.dot(p, w_ref[...], preferred_element_type=jnp.float32)


def _fused_body(g_ref, eh_ref, ei_ref, w_ref, o_ref):
    o_ref[...] = (g_ref[...]
                  - jnp.dot(eh_ref[...], w_ref[...],
                            preferred_element_type=jnp.float32)
                  + ei_ref[...])


_MAILW_BLK = 1250
_FUSE_BLK = 2000


def kernel(edge_hidden, edge_init, W, edge_index):
    src = edge_index[0].reshape(NE // G, G)
    dst = edge_index[1].reshape(NE // G, G)
    zeros = jnp.zeros((NN, D), jnp.float32)

    parts = _scatter(edge_hidden, dst, zeros)

    mail_w = pl.pallas_call(
        _mailw_body,
        grid=(NN // _MAILW_BLK,),
        in_specs=[
            pl.BlockSpec((NC, _MAILW_BLK, D), lambda i: (0, i, 0)),
            pl.BlockSpec((D, D), lambda i: (0, 0)),
        ],
        out_specs=pl.BlockSpec((_MAILW_BLK, D), lambda i: (i, 0)),
        out_shape=jax.ShapeDtypeStruct((NN, D), jnp.float32),
    )(parts, W)

    gathered = _gather(mail_w, src)

    out = pl.pallas_call(
        _fused_body,
        grid=(NE // _FUSE_BLK,),
        in_specs=[
            pl.BlockSpec((_FUSE_BLK, D), lambda i: (i, 0)),
            pl.BlockSpec((_FUSE_BLK, D), lambda i: (i, 0)),
            pl.BlockSpec((_FUSE_BLK, D), lambda i: (i, 0)),
            pl.BlockSpec((D, D), lambda i: (0, 0)),
        ],
        out_specs=pl.BlockSpec((_FUSE_BLK, D), lambda i: (i, 0)),
        out_shape=jax.ShapeDtypeStruct((NE, D), jnp.float32),
    )(gathered, edge_hidden, edge_init, W)

    return out


# speedup vs baseline: 3.3904x; 3.3904x over previous
import jax
import jax.numpy as jnp
from jax import lax
from jax.experimental import pallas as pl
from jax.experimental.pallas import tpu as pltpu
from jax.experimental.pallas import tpu_sc as plsc

NE = 320000
NN = 10000
D = 128

NC = 2
NS = 16
NW = NC * NS
EPW = NE // NW
G = 100
IROWS = EPW // G

CHS = 200
NCHS = EPW // CHS
CHG = 400
NCHG = EPW // CHG

_mesh = plsc.VectorSubcoreMesh(core_axis_name="c", subcore_axis_name="s")


def _scatter_body(eh_hbm, dst_hbm, zero_hbm, parts_hbm, idx_v, ebuf, mail_sh):
    cid = lax.axis_index("c")
    sid = lax.axis_index("s")
    wid = cid * NS + sid

    @pl.when(sid == 0)
    def _():
        pltpu.sync_copy(zero_hbm, mail_sh)

    plsc.subcore_barrier()

    pltpu.sync_copy(dst_hbm.at[wid], idx_v)

    ebase = wid * EPW

    def chunk(i, carry):
        pltpu.sync_copy(eh_hbm.at[pl.ds(ebase + i * CHS, CHS)], ebuf)
        for b in range(CHS // G):
            pltpu.sync_copy(ebuf.at[pl.ds(b * G, G)],
                            mail_sh.at[idx_v.at[i * (CHS // G) + b]],
                            add=True)
        return carry

    lax.fori_loop(0, NCHS, chunk, 0)
    plsc.subcore_barrier()

    @pl.when(sid == 0)
    def _():
        pltpu.sync_copy(mail_sh, parts_hbm.at[cid])


_scatter = pl.kernel(
    _scatter_body,
    out_type=jax.ShapeDtypeStruct((NC, NN, D), jnp.float32),
    mesh=_mesh,
    scratch_types=[
        pltpu.VMEM((IROWS, G), jnp.int32),
        pltpu.VMEM((CHS, D), jnp.float32),
        pltpu.VMEM_SHARED((NN, D), jnp.float32),
    ],
)


def _gather_body(mw_hbm, src_hbm, out_hbm, idx_v, gbuf, sem):
    cid = lax.axis_index("c")
    sid = lax.axis_index("s")
    wid = cid * NS + sid

    pltpu.sync_copy(src_hbm.at[wid], idx_v)
    ebase = wid * EPW

    def chunk(i, carry):
        descs = [
            pltpu.async_copy(mw_hbm.at[idx_v.at[i * (CHG // G) + b]],
                             gbuf.at[pl.ds(b * G, G)], sem)
            for b in range(CHG // G)
        ]
        for d in descs:
            d.wait()
        pltpu.sync_copy(gbuf, out_hbm.at[pl.ds(ebase + i * CHG, CHG)])
        return carry

    lax.fori_loop(0, NCHG, chunk, 0)


_gather = pl.kernel(
    _gather_body,
    out_type=jax.ShapeDtypeStruct((NE, D), jnp.float32),
    mesh=_mesh,
    scratch_types=[
        pltpu.VMEM((IROWS, G), jnp.int32),
        pltpu.VMEM((CHG, D), jnp.float32),
        pltpu.SemaphoreType.DMA,
    ],
)


def _mailw_body(parts_ref, w_ref, o_ref):
    p = parts_ref[0] + parts_ref[1]
    o_ref[...] = jnp.dot(p, w_ref[...], preferred_element_type=jnp.float32)


def _fused_body(g_ref, eh_ref, ei_ref, w_ref, o_ref):
    o_ref[...] = (g_ref[...]
                  - jnp.dot(eh_ref[...], w_ref[...],
                            preferred_element_type=jnp.float32)
                  + ei_ref[...])


_MAILW_BLK = 1000
_FUSE_BLK = 2000


def kernel(edge_hidden, edge_init, W, edge_index):
    src = edge_index[0].reshape(NW, IROWS, G)
    dst = edge_index[1].reshape(NW, IROWS, G)
    zeros = jnp.zeros((NN, D), jnp.float32)

    parts = _scatter(edge_hidden, dst, zeros)

    mail_w = pl.pallas_call(
        _mailw_body,
        grid=(NN // _MAILW_BLK,),
        in_specs=[
            pl.BlockSpec((NC, _MAILW_BLK, D), lambda i: (0, i, 0)),
            pl.BlockSpec((D, D), lambda i: (0, 0)),
        ],
        out_specs=pl.BlockSpec((_MAILW_BLK, D), lambda i: (i, 0)),
        out_shape=jax.ShapeDtypeStruct((NN, D), jnp.float32),
    )(parts, W)

    gathered = _gather(mail_w, src)

    out = pl.pallas_call(
        _fused_body,
        grid=(NE // _FUSE_BLK,),
        in_specs=[
            pl.BlockSpec((_FUSE_BLK, D), lambda i: (i, 0)),
            pl.BlockSpec((_FUSE_BLK, D), lambda i: (i, 0)),
            pl.BlockSpec((_FUSE_BLK, D), lambda i: (i, 0)),
            pl.BlockSpec((D, D), lambda i: (0, 0)),
        ],
        out_specs=pl.BlockSpec((_FUSE_BLK, D), lambda i: (i, 0)),
        out_shape=jax.ShapeDtypeStruct((NE, D), jnp.float32),
    )(gathered, edge_hidden, edge_init, W)

    return out
